# Initial kernel scaffold; baseline (speedup 1.0000x reference)
import jax
import jax.numpy as jnp
from jax.experimental import pallas as pl


def kernel(x_exp, feat_gate, edge_gate, cached_gcn, cached_proj, W_proj, b_proj, W1, b1, W2, b2, W_head, b_head, edge_index):
    raise NotImplementedError("write your pallas kernel here")


# speedup vs baseline: 35.3098x; 35.3098x over previous
import functools

import jax
import jax.numpy as jnp
from jax import lax
from jax.experimental import pallas as pl
from jax.experimental.pallas import tpu as pltpu
from jax.experimental.pallas import tpu_sc as plsc

N = 10000
E = 320000
DIM = 128
NC = 2
NS = 16
NW = NC * NS
NPAD = 10240
CHUNK = NPAD
EPAD = NW * CHUNK
ROWS = CHUNK // 128
GROUPS = CHUNK // 16
HALF = NPAD // 2
SUB = 2048
SUBS = 11
EC = NS * SUBS * SUB

_mesh = plsc.VectorSubcoreMesh(core_axis_name="c", subcore_axis_name="s")
_sc_params = pltpu.CompilerParams(needs_layout_passes=False)


@functools.partial(
    pl.kernel,
    mesh=_mesh,
    out_type=(
        jax.ShapeDtypeStruct((NC, NPAD), jnp.float32),
        jax.ShapeDtypeStruct((NW, NPAD), jnp.float32),
    ),
    scratch_types=[
        pltpu.VMEM((ROWS, 128), jnp.int32),
        pltpu.VMEM((ROWS, 128), jnp.int32),
        pltpu.VMEM((ROWS, 128), jnp.float32),
        pltpu.VMEM((NPAD,), jnp.float32),
        pltpu.VMEM((NPAD // NS,), jnp.float32),
        pltpu.VMEM_SHARED((NPAD,), jnp.float32),
    ],
    compiler_params=_sc_params,
)
def _pass_a(src_h, dst_h, w_h, deg_out, rel_out, dst_v, src_v, w_v,
            rel_v, z_v, deg_sh):
    cid = lax.axis_index("c")
    sid = lax.axis_index("s")
    wid = sid * NC + cid

    pltpu.sync_copy(dst_h.at[wid], dst_v)
    pltpu.sync_copy(src_h.at[wid], src_v)
    pltpu.sync_copy(w_h.at[wid], w_v)

    zero16 = jnp.zeros((16,), jnp.float32)

    def zloop(i, _):
        z_v[pl.ds(i * 16, 16)] = zero16
        return 0
    lax.fori_loop(0, (NPAD // NS) // 16, zloop, 0)

    def rloop(i, _):
        rel_v[pl.ds(i * 16, 16)] = zero16
        return 0
    lax.fori_loop(0, NPAD // 16, rloop, 0)

    pltpu.sync_copy(z_v, deg_sh.at[pl.ds(sid * (NPAD // NS), NPAD // NS)])
    plsc.subcore_barrier()

    def dloop(j, _):
        pltpu.sync_copy(w_v.at[j], deg_sh.at[dst_v.at[j]], add=True)
        return 0
    lax.fori_loop(0, ROWS, dloop, 0)

    ones16 = jnp.ones((16,), jnp.float32)

    def mloop(i, _):
        r = i // 8
        c = i % 8
        d16 = dst_v[r, pl.ds(c * 16, 16)]
        s16 = src_v[r, pl.ds(c * 16, 16)]
        plsc.store_scatter(rel_v, [s16], ones16, mask=d16 == 0)
        return 0
    lax.fori_loop(0, GROUPS, mloop, 0)

    plsc.subcore_barrier()

    @pl.when(sid == 0)
    def _():
        pltpu.sync_copy(deg_sh, deg_out.at[cid])

    pltpu.sync_copy(rel_v, rel_out.at[wid])


def _kern_b(x_ref, gate_ref, wp_ref, bp_ref, deg_ref, rel_ref,
            p_ref, dinv_ref, relc_ref):
    x = x_ref[...]
    gate = gate_ref[...]
    p_ref[...] = jnp.dot(x * gate, wp_ref[...],
                         preferred_element_type=jnp.float32,
                         precision=lax.Precision.HIGHEST) + bp_ref[...]
    deg = deg_ref[...]
    dsum = deg[0:80] + deg[80:160] + 1.0
    dinv_ref[...] = lax.rsqrt(dsum)
    rel = rel_ref[...]
    acc = rel[0:80]
    for i in range(1, NW):
        acc = acc + rel[i * 80:(i + 1) * 80]
    node = (lax.broadcasted_iota(jnp.int32, (80, 128), 0) * 128
            + lax.broadcasted_iota(jnp.int32, (80, 128), 1))
    relc_ref[...] = jnp.where((acc > 0.0) | (node == 0), 1.0, 0.0)


@functools.partial(
    pl.kernel,
    mesh=_mesh,
    out_type=(
        jax.ShapeDtypeStruct((NC, HALF, DIM), jnp.float32),
        jax.ShapeDtypeStruct((NC, NPAD), jnp.float32),
    ),
    scratch_types=[
        pltpu.VMEM((SUB,), jnp.int32),
        pltpu.VMEM((SUB,), jnp.int32),
        pltpu.VMEM((SUB,), jnp.float32),
        pltpu.VMEM((NPAD,), jnp.float32),
        pltpu.VMEM((NPAD,), jnp.float32),
        pltpu.VMEM((SUB,), jnp.int32),
        pltpu.VMEM((SUB,), jnp.int32),
        pltpu.VMEM((SUB,), jnp.float32),
        pltpu.VMEM((16,), jnp.float32),
        pltpu.VMEM((16, DIM), jnp.float32),
        pltpu.VMEM((16, DIM), jnp.float32),
        pltpu.VMEM((16,), jnp.float32),
        pltpu.VMEM((16, DIM), jnp.float32),
        pltpu.VMEM((NPAD // NS,), jnp.float32),
        pltpu.VMEM_SHARED((HALF, DIM), jnp.float32),
        pltpu.VMEM_SHARED((NPAD,), jnp.float32),
        pltpu.SemaphoreType.DMA,
    ],
    compiler_params=_sc_params,
)
def _pass_c(src_h, dst_h, w_h, dinv_h, rel_h, p_h,
            a1_out, beta_out,
            src_v, dst_v, w_v, dinv_v, rel_v, sel_s, sel_d, sel_n,
            normg, prow, srow, bval, zrow, z_v, a1_sh, beta_sh, sem):
    cid = lax.axis_index("c")
    sid = lax.axis_index("s")
    lo = cid * HALF

    pltpu.sync_copy(dinv_h, dinv_v)
    pltpu.sync_copy(rel_h, rel_v)

    zero16 = jnp.zeros((16,), jnp.float32)

    def z1(i, _):
        r = i // 8
        c = i % 8
        zrow[r, pl.ds(c * 16, 16)] = zero16
        return 0
    lax.fori_loop(0, 16 * 8, z1, 0)

    def z2(i, _):
        z_v[pl.ds(i * 16, 16)] = zero16
        return 0
    lax.fori_loop(0, (NPAD // NS) // 16, z2, 0)

    def z3(j, _):
        pltpu.sync_copy(zrow, a1_sh.at[pl.ds(sid * (HALF // NS) + j * 16, 16)])
        return 0
    lax.fori_loop(0, (HALF // NS) // 16, z3, 0)
    pltpu.sync_copy(z_v, beta_sh.at[pl.ds(sid * (NPAD // NS), NPAD // NS)])
    plsc.subcore_barrier()

    hi = lo + HALF
    lanes = lax.iota(jnp.int32, 16)

    def sub_loop(sub, _):
        pltpu.sync_copy(src_h.at[sid, sub], src_v)
        pltpu.sync_copy(dst_h.at[sid, sub], dst_v)
        pltpu.sync_copy(w_h.at[sid, sub], w_v)

        def sel_loop(i, off):
            s16 = src_v[pl.ds(i * 16, 16)]
            d16 = dst_v[pl.ds(i * 16, 16)]
            w16 = w_v[pl.ds(i * 16, 16)]
            dv_s = plsc.load_gather(dinv_v, [s16])
            dv_d = plsc.load_gather(dinv_v, [d16])
            r_d = plsc.load_gather(rel_v, [d16])
            n16 = dv_s * w16 * dv_d
            m = (r_d > 0.0) & (d16 >= lo) & (d16 < hi)
            plsc.store_compressed(sel_s.at[pl.ds(off, 16)], s16, mask=m)
            plsc.store_compressed(sel_d.at[pl.ds(off, 16)], d16, mask=m)
            plsc.store_compressed(sel_n.at[pl.ds(off, 16)], n16, mask=m)
            return off + jnp.sum(m.astype(jnp.int32))
        count = lax.fori_loop(0, SUB // 16, sel_loop, jnp.int32(0))

        def grp(j, _):
            base = j * 16
            s16 = sel_s[pl.ds(base, 16)]
            d16 = sel_d[pl.ds(base, 16)]
            n16 = sel_n[pl.ds(base, 16)]
            valid = (lanes + base) < count
            s16 = jnp.where(valid, s16, 0)
            d16c = jnp.where(valid, d16 - lo, 0)
            n16 = jnp.where(valid, n16, 0.0)
            pltpu.async_copy(p_h.at[s16], prow, sem).wait()
            for rr in range(16):
                nb = jnp.take_along_axis(
                    n16, jnp.full((16,), rr, jnp.int32), axis=0,
                    mode="promise_in_bounds")
                for cc in range(8):
                    srow[rr, pl.ds(cc * 16, 16)] = (
                        prow[rr, pl.ds(cc * 16, 16)] * nb)
            pltpu.sync_copy(srow, a1_sh.at[d16c], add=True)
            m0 = (d16 == 0) & valid
            bval[...] = jnp.where(m0, n16, 0.0)
            b_idx = jnp.where(m0, s16, 0)
            pltpu.sync_copy(bval, beta_sh.at[b_idx], add=True)
            return 0
        lax.fori_loop(0, (count + 15) // 16, grp, 0)
        return 0
    lax.fori_loop(0, SUBS, sub_loop, 0)

    plsc.subcore_barrier()

    @pl.when(sid == 0)
    def _():
        pltpu.sync_copy(a1_sh, a1_out.at[cid])
        pltpu.sync_copy(beta_sh, beta_out.at[cid])


def _kern_d(a1_ref, beta_ref, p0_ref, cg_ref, cp_ref,
            w1_ref, b1_ref, w2_ref, b2_ref, wh_ref, bh_ref, out_ref):
    z = a1_ref[...]
    h1 = jnp.maximum(
        jnp.dot(z, w1_ref[...], preferred_element_type=jnp.float32,
                precision=lax.Precision.HIGHEST) + b1_ref[...], 0.0)
    beta = beta_ref[...]
    bsum = beta[0] + beta[1]
    m2 = jnp.dot(bsum, h1, preferred_element_type=jnp.float32,
                 precision=lax.Precision.HIGHEST)
    h2 = jnp.maximum(
        jnp.dot(m2, w2_ref[...], preferred_element_type=jnp.float32,
                precision=lax.Precision.HIGHEST) + b2_ref[...], 0.0)
    cg = cg_ref[...]
    cp = cp_ref[...]
    g = (cg[:, 0:256] + cg[:, 256:512] +
         jnp.concatenate([h1[0:1, :], h2], axis=1)) / 3.0
    r = (cp[:, 0:128] + cp[:, 128:256] + p0_ref[...]) / 3.0
    feats = jnp.concatenate([g, r], axis=1)
    out_ref[...] = jnp.dot(feats, wh_ref[...],
                           preferred_element_type=jnp.float32,
                           precision=lax.Precision.HIGHEST) + bh_ref[0, 0]


def kernel(x_exp, feat_gate, edge_gate, cached_gcn, cached_proj,
           W_proj, b_proj, W1, b1, W2, b2, W_head, b_head, edge_index):
    pad = EPAD - E
    src = jnp.pad(edge_index[0], (0, pad)).reshape(NW, ROWS, 128)
    dst = jnp.pad(edge_index[1], (0, pad),
                  constant_values=NPAD - 1).reshape(NW, ROWS, 128)
    w = jnp.pad(edge_gate, (0, pad)).reshape(NW, ROWS, 128)

    deg_parts, rel_parts = _pass_a(src, dst, w)

    x_pad = jnp.pad(x_exp, ((0, NPAD - N), (0, 0)))
    p, dinv80, rel80 = pl.pallas_call(
        _kern_b,
        out_shape=(
            jax.ShapeDtypeStruct((NPAD, DIM), jnp.float32),
            jax.ShapeDtypeStruct((80, 128), jnp.float32),
            jax.ShapeDtypeStruct((80, 128), jnp.float32),
        ),
    )(x_pad, feat_gate.reshape(1, DIM), W_proj, b_proj.reshape(1, DIM),
      deg_parts.reshape(NC * 80, 128), rel_parts.reshape(NW * 80, 128))

    dinv_flat = dinv80.reshape(NPAD)
    rel_flat = rel80.reshape(NPAD)

    pad2 = EC - E - N
    loops = jnp.arange(N, dtype=edge_index.dtype)
    src_c = jnp.concatenate(
        [edge_index[0], loops, jnp.zeros((pad2,), edge_index.dtype)]
    ).reshape(NS, SUBS, SUB)
    dst_c = jnp.concatenate(
        [edge_index[1], loops,
         jnp.full((pad2,), NPAD - 1, edge_index.dtype)]
    ).reshape(NS, SUBS, SUB)
    w_c = jnp.concatenate(
        [edge_gate, jnp.ones((N,), jnp.float32),
         jnp.zeros((pad2,), jnp.float32)]
    ).reshape(NS, SUBS, SUB)
    a1_halves, beta_parts = _pass_c(src_c, dst_c, w_c, dinv_flat, rel_flat, p)
    a1 = a1_halves.reshape(NPAD, DIM)

    p0 = lax.slice(p, (0, 0), (1, DIM))

    pred = pl.pallas_call(
        _kern_d,
        out_shape=jax.ShapeDtypeStruct((1, 1), jnp.float32),
    )(a1, beta_parts.reshape(NC, 1, NPAD), p0,
      cached_gcn.reshape(1, 512), cached_proj.reshape(1, 256),
      W1, b1.reshape(1, DIM), W2, b2.reshape(1, DIM),
      W_head, b_head.reshape(1, 1))

    return pred.reshape(())
